# baseline (speedup 1.0000x reference)
import jax
import jax.numpy as jnp
from jax import lax
from jax.experimental import pallas as pl
from jax.experimental.pallas import tpu as pltpu
from jax.experimental.pallas import tpu_sc as plsc

B = 1024
D = 512
N_PREFIX = 2
N_MIDDLE = 2
N_ATTR = 16
N_SUFFIX = 56
T = N_PREFIX + 1 + N_MIDDLE + N_ATTR + N_SUFFIX
N_CONST = N_PREFIX + N_MIDDLE + N_SUFFIX

OFF_S = 2
OFF_ATTR = 5
OFF_SUF = 21

_info = plsc.get_sparse_core_info()
_NC = _info.num_cores
_NS = _info.num_subcores
NW = _NC * _NS

QB = 256
NQ = B // QB
M_TOTAL = T * NQ
GR = 64


def _body(s_ref, attr_ref, const_ref, out_ref,
          rep_v, g0, g1, idx_v, osem, gsem, asem0, asem1):
    cid = lax.axis_index("c")
    sid = lax.axis_index("s")
    wid = sid * _NC + cid
    m0 = wid * M_TOTAL // NW
    m1 = (wid + 1) * M_TOTAL // NW
    gbufs = (g0, g1)

    def do_const(t, b0, last_r):
        r = jnp.where(t < OFF_S, t,
                      jnp.where(t < OFF_ATTR, t - 1,
                                t - OFF_SUF + N_PREFIX + N_MIDDLE))

        def refill():
            for i in range(32):
                pltpu.make_async_copy(const_ref.at[r], rep_v.at[i], gsem).start()
            for i in range(32):
                pltpu.make_async_copy(const_ref.at[r], rep_v.at[i], gsem).wait()
            return r

        lax.cond(r != last_r, refill, lambda: r)
        dst0 = t * B + b0
        for k in range(QB // 32):
            pltpu.make_async_copy(
                rep_v, out_ref.at[pl.ds(dst0 + 32 * k, 32)], osem).start()
        for k in range(QB // 32):
            pltpu.make_async_copy(
                rep_v, out_ref.at[pl.ds(dst0 + 32 * k, 32)], osem).wait()
        return r

    def do_s(b0):
        pltpu.sync_copy(
            s_ref.at[pl.ds(b0, QB)], out_ref.at[pl.ds(OFF_S * B + b0, QB)])

    def do_attr(t, b0):
        j = t - OFF_ATTR
        asems = (asem0, asem1)
        for p in range(QB // GR):
            g = gbufs[p % 2]
            sem = asems[p % 2]
            if p >= 2:
                pltpu.make_async_copy(
                    g, out_ref.at[pl.ds(t * B + b0 + GR * (p - 2), GR)],
                    sem).wait()
            for q in range(GR):
                pltpu.make_async_copy(
                    attr_ref.at[(b0 + GR * p + q) * N_ATTR + j],
                    g.at[q], gsem).start()
            for q in range(GR):
                pltpu.make_async_copy(
                    attr_ref.at[(b0 + GR * p + q) * N_ATTR + j],
                    g.at[q], gsem).wait()
            pltpu.make_async_copy(
                g, out_ref.at[pl.ds(t * B + b0 + GR * p, GR)], sem).start()
        for p in range(2, 4):
            pltpu.make_async_copy(
                gbufs[p % 2],
                out_ref.at[pl.ds(t * B + b0 + GR * p, GR)],
                asems[p % 2]).wait()

    def step(m, last_r):
        t = m // NQ
        b0 = (m % NQ) * QB

        def s_br():
            do_s(b0)
            return last_r

        def attr_br():
            do_attr(t, b0)
            return last_r

        def const_br():
            return do_const(t, b0, last_r)

        return lax.cond(
            t == OFF_S,
            s_br,
            lambda: lax.cond(
                jnp.logical_and(t >= OFF_ATTR, t < OFF_SUF),
                attr_br, const_br),
        )

    lax.fori_loop(m0, m1, step, jnp.int32(-1))


def kernel(s_star, attr_tokens, token_prefix, token_middle, token_suffix):
    consts = jnp.concatenate(
        [token_prefix.reshape(N_PREFIX, D),
         token_middle.reshape(N_MIDDLE, D),
         token_suffix.reshape(N_SUFFIX, D)], axis=0)
    mesh = plsc.VectorSubcoreMesh(core_axis_name="c", subcore_axis_name="s")
    f = pl.kernel(
        _body,
        out_type=jax.ShapeDtypeStruct((T * B, D), jnp.float32),
        mesh=mesh,
        scratch_types=[
            pltpu.VMEM((32, D), jnp.float32),
            pltpu.VMEM((GR, D), jnp.float32),
            pltpu.VMEM((GR, D), jnp.float32),
            pltpu.VMEM((GR,), jnp.int32),
            pltpu.SemaphoreType.DMA,
            pltpu.SemaphoreType.DMA,
            pltpu.SemaphoreType.DMA,
            pltpu.SemaphoreType.DMA,
        ],
    )
    flat = f(s_star, attr_tokens.reshape(B * N_ATTR, D), consts)
    return flat.reshape(T, B, D).transpose(1, 0, 2)


# speedup vs baseline: 1.0559x; 1.0559x over previous
import jax
import jax.numpy as jnp
from jax import lax
from jax.experimental import pallas as pl
from jax.experimental.pallas import tpu as pltpu
from jax.experimental.pallas import tpu_sc as plsc

B = 1024
D = 512
N_PREFIX = 2
N_MIDDLE = 2
N_ATTR = 16
N_SUFFIX = 56
T = N_PREFIX + 1 + N_MIDDLE + N_ATTR + N_SUFFIX

OFF_S = 2
OFF_ATTR = 5
OFF_SUF = 21

_info = plsc.get_sparse_core_info()
_NC = _info.num_cores
_NS = _info.num_subcores
NW = _NC * _NS

QB = 256
NQ = B // QB
M_TOTAL = T * NQ
GR = 64


def _body(s_ref, attr_ref, const_ref, out_ref,
          rep_v, g0, g1, idx_v, osem, gsem, ssem, asem0, asem1):
    cid = lax.axis_index("c")
    sid = lax.axis_index("s")
    wid = sid * _NC + cid
    m0 = wid * M_TOTAL // NW
    m1 = (wid + 1) * M_TOTAL // NW
    gbufs = (g0, g1)
    asems = (asem0, asem1)

    def drain_pending(n):
        lax.fori_loop(
            0, n,
            lambda _, c: (pltpu.make_async_copy(
                rep_v, out_ref.at[pl.ds(0, 32)], osem).wait(), c)[1],
            0)

    def do_const(t, b0, last_r, pending):
        r = jnp.where(t < OFF_S, t,
                      jnp.where(t < OFF_ATTR, t - 1,
                                t - OFF_SUF + N_PREFIX + N_MIDDLE))

        def refill():
            drain_pending(pending)
            for i in range(32):
                pltpu.make_async_copy(const_ref.at[r], rep_v.at[i], gsem).start()
            for i in range(32):
                pltpu.make_async_copy(const_ref.at[r], rep_v.at[i], gsem).wait()
            return jnp.int32(0)

        pend = lax.cond(r != last_r, refill, lambda: pending)
        dst0 = t * B + b0
        for k in range(QB // 32):
            pltpu.make_async_copy(
                rep_v, out_ref.at[pl.ds(dst0 + 32 * k, 32)], osem).start()
        return r, pend + QB // 32

    def do_s(b0):
        pltpu.make_async_copy(
            s_ref.at[pl.ds(b0, QB)], out_ref.at[pl.ds(OFF_S * B + b0, QB)],
            ssem).start()

    def do_attr(t, b0):
        j = t - OFF_ATTR
        it = lax.iota(jnp.int32, 16)
        for p in range(QB // GR):
            g = gbufs[p % 2]
            sem = asems[p % 2]
            if p >= 2:
                pltpu.make_async_copy(
                    g, out_ref.at[pl.ds(t * B + b0 + GR * (p - 2), GR)],
                    sem).wait()
            for k in range(GR // 16):
                idx_v[pl.ds(16 * k, 16)] = (
                    it + (b0 + GR * p + 16 * k)) * N_ATTR + j
            pltpu.async_copy(attr_ref.at[idx_v], g, gsem).wait()
            pltpu.make_async_copy(
                g, out_ref.at[pl.ds(t * B + b0 + GR * p, GR)], sem).start()
        for p in range(2, 4):
            pltpu.make_async_copy(
                gbufs[p % 2],
                out_ref.at[pl.ds(t * B + b0 + GR * p, GR)],
                asems[p % 2]).wait()

    def step(m, carry):
        last_r, pending, n_s = carry
        t = m // NQ
        b0 = (m % NQ) * QB

        def s_br():
            do_s(b0)
            return last_r, pending, n_s + 1

        def attr_br():
            do_attr(t, b0)
            return last_r, pending, n_s

        def const_br():
            r, pend = do_const(t, b0, last_r, pending)
            return r, pend, n_s

        return lax.cond(
            t == OFF_S,
            s_br,
            lambda: lax.cond(
                jnp.logical_and(t >= OFF_ATTR, t < OFF_SUF),
                attr_br, const_br),
        )

    _, pending, n_s = lax.fori_loop(
        m0, m1, step, (jnp.int32(-1), jnp.int32(0), jnp.int32(0)))
    drain_pending(pending)
    lax.fori_loop(
        0, n_s,
        lambda _, c: (pltpu.make_async_copy(
            s_ref.at[pl.ds(0, QB)], out_ref.at[pl.ds(0, QB)],
            ssem).wait(), c)[1],
        0)


def kernel(s_star, attr_tokens, token_prefix, token_middle, token_suffix):
    consts = jnp.concatenate(
        [token_prefix.reshape(N_PREFIX, D),
         token_middle.reshape(N_MIDDLE, D),
         token_suffix.reshape(N_SUFFIX, D)], axis=0)
    mesh = plsc.VectorSubcoreMesh(core_axis_name="c", subcore_axis_name="s")
    f = pl.kernel(
        _body,
        out_type=jax.ShapeDtypeStruct((T * B, D), jnp.float32),
        mesh=mesh,
        scratch_types=[
            pltpu.VMEM((32, D), jnp.float32),
            pltpu.VMEM((GR, D), jnp.float32),
            pltpu.VMEM((GR, D), jnp.float32),
            pltpu.VMEM((GR,), jnp.int32),
            pltpu.SemaphoreType.DMA,
            pltpu.SemaphoreType.DMA,
            pltpu.SemaphoreType.DMA,
            pltpu.SemaphoreType.DMA,
            pltpu.SemaphoreType.DMA,
        ],
    )
    flat = f(s_star, attr_tokens.reshape(B * N_ATTR, D), consts)
    return flat.reshape(T, B, D).transpose(1, 0, 2)


# speedup vs baseline: 1.1123x; 1.0534x over previous
import jax
import jax.numpy as jnp
from jax import lax
from jax.experimental import pallas as pl
from jax.experimental.pallas import tpu as pltpu
from jax.experimental.pallas import tpu_sc as plsc

B = 1024
D = 512
N_PREFIX = 2
N_MIDDLE = 2
N_ATTR = 16
N_SUFFIX = 56
T = N_PREFIX + 1 + N_MIDDLE + N_ATTR + N_SUFFIX

OFF_S = 2
OFF_ATTR = 5
OFF_SUF = 21

_info = plsc.get_sparse_core_info()
_NC = _info.num_cores
_NS = _info.num_subcores
NW = _NC * _NS

QB = 256
NQ = B // QB
M_TOTAL = T * NQ
GR = 64


def _body(s_ref, attr_ref, const_ref, out_ref,
          rep_v, g0, g1, idx0, idx1, osem, gsem, ssem, asem0, asem1):
    cid = lax.axis_index("c")
    sid = lax.axis_index("s")
    wid = sid * _NC + cid
    m0 = wid * M_TOTAL // NW
    m1 = (wid + 1) * M_TOTAL // NW
    gbufs = (g0, g1)
    asems = (asem0, asem1)

    def drain_pending(n):
        lax.fori_loop(
            0, n,
            lambda _, c: (pltpu.make_async_copy(
                rep_v, out_ref.at[pl.ds(0, 32)], osem).wait(), c)[1],
            0)

    def do_const(t, b0, last_r, pending):
        r = jnp.where(t < OFF_S, t,
                      jnp.where(t < OFF_ATTR, t - 1,
                                t - OFF_SUF + N_PREFIX + N_MIDDLE))

        def refill():
            drain_pending(pending)
            for i in range(32):
                pltpu.make_async_copy(const_ref.at[r], rep_v.at[i], gsem).start()
            for i in range(32):
                pltpu.make_async_copy(const_ref.at[r], rep_v.at[i], gsem).wait()
            return jnp.int32(0)

        pend = lax.cond(r != last_r, refill, lambda: pending)
        dst0 = t * B + b0
        for k in range(QB // 32):
            pltpu.make_async_copy(
                rep_v, out_ref.at[pl.ds(dst0 + 32 * k, 32)], osem).start()
        return r, pend + QB // 32

    def do_s(b0):
        pltpu.make_async_copy(
            s_ref.at[pl.ds(b0, QB)], out_ref.at[pl.ds(OFF_S * B + b0, QB)],
            ssem).start()

    def do_attr(t, b0):
        j = t - OFF_ATTR
        it = lax.iota(jnp.int32, 16)
        idxs = (idx0, idx1)

        def sg(p):
            iv = idxs[p % 2]
            for k in range(GR // 16):
                iv[pl.ds(16 * k, 16)] = (
                    it + (b0 + GR * p + 16 * k)) * N_ATTR + j
            pltpu.make_async_copy(attr_ref.at[iv], gbufs[p % 2], gsem).start()

        def wg(p):
            pltpu.make_async_copy(
                attr_ref.at[idxs[p % 2]], gbufs[p % 2], gsem).wait()

        def so(p):
            pltpu.make_async_copy(
                gbufs[p % 2], out_ref.at[pl.ds(t * B + b0 + GR * p, GR)],
                asems[p % 2]).start()

        def wo(p):
            pltpu.make_async_copy(
                gbufs[p % 2], out_ref.at[pl.ds(t * B + b0 + GR * p, GR)],
                asems[p % 2]).wait()

        sg(0)
        sg(1)
        wg(0)
        so(0)
        wg(1)
        so(1)
        wo(0)
        sg(2)
        wg(2)
        so(2)
        wo(1)
        sg(3)
        wg(3)
        so(3)
        wo(2)
        wo(3)

    def step(m, carry):
        last_r, pending, n_s = carry
        t = m // NQ
        b0 = (m % NQ) * QB

        def s_br():
            do_s(b0)
            return last_r, pending, n_s + 1

        def attr_br():
            do_attr(t, b0)
            return last_r, pending, n_s

        def const_br():
            r, pend = do_const(t, b0, last_r, pending)
            return r, pend, n_s

        return lax.cond(
            t == OFF_S,
            s_br,
            lambda: lax.cond(
                jnp.logical_and(t >= OFF_ATTR, t < OFF_SUF),
                attr_br, const_br),
        )

    _, pending, n_s = lax.fori_loop(
        m0, m1, step, (jnp.int32(-1), jnp.int32(0), jnp.int32(0)))
    drain_pending(pending)
    lax.fori_loop(
        0, n_s,
        lambda _, c: (pltpu.make_async_copy(
            s_ref.at[pl.ds(0, QB)], out_ref.at[pl.ds(0, QB)],
            ssem).wait(), c)[1],
        0)


def kernel(s_star, attr_tokens, token_prefix, token_middle, token_suffix):
    consts = jnp.concatenate(
        [token_prefix.reshape(N_PREFIX, D),
         token_middle.reshape(N_MIDDLE, D),
         token_suffix.reshape(N_SUFFIX, D)], axis=0)
    mesh = plsc.VectorSubcoreMesh(core_axis_name="c", subcore_axis_name="s")
    f = pl.kernel(
        _body,
        out_type=jax.ShapeDtypeStruct((T * B, D), jnp.float32),
        mesh=mesh,
        scratch_types=[
            pltpu.VMEM((32, D), jnp.float32),
            pltpu.VMEM((GR, D), jnp.float32),
            pltpu.VMEM((GR, D), jnp.float32),
            pltpu.VMEM((GR,), jnp.int32),
            pltpu.VMEM((GR,), jnp.int32),
            pltpu.SemaphoreType.DMA,
            pltpu.SemaphoreType.DMA,
            pltpu.SemaphoreType.DMA,
            pltpu.SemaphoreType.DMA,
            pltpu.SemaphoreType.DMA,
        ],
    )
    flat = f(s_star, attr_tokens.reshape(B * N_ATTR, D), consts)
    return flat.reshape(T, B, D).transpose(1, 0, 2)
